# baseline (speedup 1.0000x reference)
import math

import jax
import jax.numpy as jnp
from jax.experimental import pallas as pl

_C = math.sqrt(2.0 / math.pi)
_ROWS = 8192
_COLS = 4096
_BLOCK_ROWS = 256


def _gelu_block(x_ref, o_ref):
    x = x_ref[...]
    inner = _C * (x + 0.044715 * (x * x * x))
    o_ref[...] = 0.5 * x * (1.0 + jnp.tanh(inner))


def kernel(x, log_k_blend):
    del log_k_blend
    x2 = x.reshape(_ROWS, _COLS)
    out = pl.pallas_call(
        _gelu_block,
        grid=(_ROWS // _BLOCK_ROWS,),
        in_specs=[pl.BlockSpec((_BLOCK_ROWS, _COLS), lambda i: (i, 0))],
        out_specs=pl.BlockSpec((_BLOCK_ROWS, _COLS), lambda i: (i, 0)),
        out_shape=jax.ShapeDtypeStruct((_ROWS, _COLS), jnp.float32),
    )(x2)
    return out.reshape(x.shape)


# speedup vs baseline: 1.0100x; 1.0100x over previous
import math

import jax
import jax.numpy as jnp
from jax.experimental import pallas as pl
from jax.experimental.pallas import tpu as pltpu

_C = math.sqrt(2.0 / math.pi)
_A = _C * 0.044715
_ROWS = 8192
_COLS = 4096
_BLOCK_ROWS = 256


def _gelu_block(x_ref, o_ref):
    x = x_ref[...]
    u = x * x
    z = x * (_C + _A * u)
    h = 0.5 * x
    t = jnp.tanh(z)
    o_ref[...] = h + h * t


def kernel(x, log_k_blend):
    del log_k_blend
    x2 = x.reshape(_ROWS, _COLS)
    out = pl.pallas_call(
        _gelu_block,
        grid=(_ROWS // _BLOCK_ROWS,),
        in_specs=[pl.BlockSpec((_BLOCK_ROWS, _COLS), lambda i: (i, 0))],
        out_specs=pl.BlockSpec((_BLOCK_ROWS, _COLS), lambda i: (i, 0)),
        out_shape=jax.ShapeDtypeStruct((_ROWS, _COLS), jnp.float32),
        compiler_params=pltpu.CompilerParams(
            dimension_semantics=("parallel",),
        ),
    )(x2)
    return out.reshape(x.shape)


# speedup vs baseline: 1.0315x; 1.0212x over previous
import math

import jax
import jax.numpy as jnp
from jax.experimental import pallas as pl
from jax.experimental.pallas import tpu as pltpu

_C = math.sqrt(2.0 / math.pi)
_A = _C * 0.044715
_ROWS = 8192
_COLS = 4096
_BLOCK_ROWS = 512


def _gelu_block(x_ref, o_ref):
    x = x_ref[...]
    u = x * x
    z = x * (_C + _A * u)
    h = 0.5 * x
    t = jnp.tanh(z)
    o_ref[...] = h + h * t


def kernel(x, log_k_blend):
    del log_k_blend
    x2 = x.reshape(_ROWS, _COLS)
    out = pl.pallas_call(
        _gelu_block,
        grid=(_ROWS // _BLOCK_ROWS,),
        in_specs=[pl.BlockSpec((_BLOCK_ROWS, _COLS), lambda i: (i, 0))],
        out_specs=pl.BlockSpec((_BLOCK_ROWS, _COLS), lambda i: (i, 0)),
        out_shape=jax.ShapeDtypeStruct((_ROWS, _COLS), jnp.float32),
        compiler_params=pltpu.CompilerParams(
            dimension_semantics=("parallel",),
            vmem_limit_bytes=120 * 1024 * 1024,
        ),
    )(x2)
    return out.reshape(x.shape)
